# baseline (speedup 1.0000x reference)
import jax
import jax.numpy as jnp
from jax.experimental import pallas as pl
from jax.experimental.pallas import tpu as pltpu

_BM = 1024
_ABUF = 4
_PRIME = 2
_NSUB = 4
_OBUF = 4


def _dot_nt(a, w):
    return jax.lax.dot_general(
        a,
        w,
        dimension_numbers=(((1,), (1,)), ((), ())),
        preferred_element_type=jnp.float32,
    )


def _in_copy(a_hbm, a_bufs, in_sem, i):
    return pltpu.make_async_copy(
        a_hbm.at[pl.ds(i * _BM, _BM), :], a_bufs[i % _ABUF], in_sem.at[i % _ABUF]
    )


def _out_wait(o_bufs, o_hbm, out_sem, i):
    sub = _BM // _NSUB
    for h in range(_NSUB):
        pltpu.make_async_copy(
            o_bufs[i % _OBUF].at[pl.ds(h * sub, sub), :],
            o_hbm.at[pl.ds(i * _BM + h * sub, sub), :],
            out_sem.at[i % _OBUF],
        ).wait()


def _pipeline_body(*refs):
    a_hbm, w_hbm, b_ref, o_hbm = refs[:4]
    a_bufs = list(refs[4:4 + _ABUF])
    wf, wbf = refs[4 + _ABUF:6 + _ABUF]
    o_bufs = list(refs[6 + _ABUF:6 + _ABUF + _OBUF])
    in_sem, out_sem, w_sem = refs[6 + _ABUF + _OBUF:]
    n_steps = a_hbm.shape[0] // _BM

    w_copy = pltpu.make_async_copy(w_hbm, wf, w_sem)
    w_copy.start()
    for i in range(min(_PRIME, n_steps)):
        _in_copy(a_hbm, a_bufs, in_sem, i).start()
    w_copy.wait()
    wbf[...] = wf[...].astype(jnp.bfloat16)

    for i in range(n_steps):
        _in_copy(a_hbm, a_bufs, in_sem, i).wait()
        if i >= _OBUF:
            _out_wait(o_bufs, o_hbm, out_sem, i - _OBUF)
        ob = o_bufs[i % _OBUF]
        ab = a_bufs[i % _ABUF]
        sub = _BM // _NSUB
        for h in range(_NSUB):
            ob[h * sub:(h + 1) * sub, :] = (
                _dot_nt(ab[h * sub:(h + 1) * sub, :].astype(jnp.bfloat16), wbf[...])
                + b_ref[...]
            )
            pltpu.make_async_copy(
                ob.at[pl.ds(h * sub, sub), :],
                o_hbm.at[pl.ds(i * _BM + h * sub, sub), :],
                out_sem.at[i % _OBUF],
            ).start()
        if i + _PRIME < n_steps:
            _in_copy(a_hbm, a_bufs, in_sem, i + _PRIME).start()

    for i in range(max(n_steps - _OBUF, 0), n_steps):
        _out_wait(o_bufs, o_hbm, out_sem, i)


def kernel(body_output, W, b):
    M, K = body_output.shape
    N = W.shape[0]
    b2d = b.reshape(1, N)
    return pl.pallas_call(
        _pipeline_body,
        in_specs=[
            pl.BlockSpec(memory_space=pltpu.MemorySpace.HBM),
            pl.BlockSpec(memory_space=pltpu.MemorySpace.HBM),
            pl.BlockSpec(memory_space=pltpu.MemorySpace.VMEM),
        ],
        out_specs=pl.BlockSpec(memory_space=pltpu.MemorySpace.HBM),
        out_shape=jax.ShapeDtypeStruct((M, N), jnp.float32),
        scratch_shapes=(
            [pltpu.VMEM((_BM, K), jnp.float32)] * _ABUF
            + [
                pltpu.VMEM((N, K), jnp.float32),
                pltpu.VMEM((N, K), jnp.bfloat16),
            ]
            + [pltpu.VMEM((_BM, N), jnp.float32)] * _OBUF
            + [
                pltpu.SemaphoreType.DMA((_ABUF,)),
                pltpu.SemaphoreType.DMA((_OBUF,)),
                pltpu.SemaphoreType.DMA,
            ]
        ),
        compiler_params=pltpu.CompilerParams(
            vmem_limit_bytes=100 * 1024 * 1024,
        ),
    )(body_output, W, b2d)


# speedup vs baseline: 1.0144x; 1.0144x over previous
import jax
import jax.numpy as jnp
from jax.experimental import pallas as pl
from jax.experimental.pallas import tpu as pltpu

_BM = 1024
_ABUF = 3
_PRIME = 2
_OBUF = 3


def _dot_nt(a, w):
    return jax.lax.dot_general(
        a,
        w,
        dimension_numbers=(((1,), (1,)), ((), ())),
        preferred_element_type=jnp.float32,
    )


def _in_copy(a_hbm, a_bufs, in_sem, i):
    return pltpu.make_async_copy(
        a_hbm.at[pl.ds(i * _BM, _BM), :], a_bufs[i % _ABUF], in_sem.at[i % _ABUF]
    )


def _out_wait(o_bufs, o_hbm, out_sem, i):
    half = _BM // 2
    for h in range(2):
        pltpu.make_async_copy(
            o_bufs[i % _OBUF].at[pl.ds(h * half, half), :],
            o_hbm.at[pl.ds(i * _BM + h * half, half), :],
            out_sem.at[i % _OBUF],
        ).wait()


def _pipeline_body(*refs):
    a_hbm, w_hbm, b_ref, o_hbm = refs[:4]
    a_bufs = list(refs[4:4 + _ABUF])
    wf, wbf = refs[4 + _ABUF:6 + _ABUF]
    o_bufs = list(refs[6 + _ABUF:6 + _ABUF + _OBUF])
    in_sem, out_sem, w_sem = refs[6 + _ABUF + _OBUF:]
    n_steps = a_hbm.shape[0] // _BM

    w_copy = pltpu.make_async_copy(w_hbm, wf, w_sem)
    w_copy.start()
    for i in range(min(_PRIME, n_steps)):
        _in_copy(a_hbm, a_bufs, in_sem, i).start()
    w_copy.wait()
    wbf[...] = wf[...].astype(jnp.bfloat16)

    for i in range(n_steps):
        _in_copy(a_hbm, a_bufs, in_sem, i).wait()
        if i >= _OBUF:
            _out_wait(o_bufs, o_hbm, out_sem, i - _OBUF)
        ob = o_bufs[i % _OBUF]
        ab = a_bufs[i % _ABUF]
        half = _BM // 2
        ob[: half, :] = (
            _dot_nt(ab[: half, :].astype(jnp.bfloat16), wbf[...]) + b_ref[...]
        )
        pltpu.make_async_copy(
            ob.at[pl.ds(0, half), :],
            o_hbm.at[pl.ds(i * _BM, half), :],
            out_sem.at[i % _OBUF],
        ).start()
        ob[half:, :] = (
            _dot_nt(ab[half:, :].astype(jnp.bfloat16), wbf[...]) + b_ref[...]
        )
        pltpu.make_async_copy(
            ob.at[pl.ds(half, half), :],
            o_hbm.at[pl.ds(i * _BM + half, half), :],
            out_sem.at[i % _OBUF],
        ).start()
        if i + _PRIME < n_steps:
            _in_copy(a_hbm, a_bufs, in_sem, i + _PRIME).start()

    for i in range(max(n_steps - _OBUF, 0), n_steps):
        _out_wait(o_bufs, o_hbm, out_sem, i)


def kernel(body_output, W, b):
    M, K = body_output.shape
    N = W.shape[0]
    b2d = b.reshape(1, N)
    return pl.pallas_call(
        _pipeline_body,
        in_specs=[
            pl.BlockSpec(memory_space=pltpu.MemorySpace.HBM),
            pl.BlockSpec(memory_space=pltpu.MemorySpace.HBM),
            pl.BlockSpec(memory_space=pltpu.MemorySpace.VMEM),
        ],
        out_specs=pl.BlockSpec(memory_space=pltpu.MemorySpace.HBM),
        out_shape=jax.ShapeDtypeStruct((M, N), jnp.float32),
        scratch_shapes=(
            [pltpu.VMEM((_BM, K), jnp.float32)] * _ABUF
            + [
                pltpu.VMEM((N, K), jnp.float32),
                pltpu.VMEM((N, K), jnp.bfloat16),
            ]
            + [pltpu.VMEM((_BM, N), jnp.float32)] * _OBUF
            + [
                pltpu.SemaphoreType.DMA((_ABUF,)),
                pltpu.SemaphoreType.DMA((_OBUF,)),
                pltpu.SemaphoreType.DMA,
            ]
        ),
        compiler_params=pltpu.CompilerParams(
            vmem_limit_bytes=100 * 1024 * 1024,
        ),
    )(body_output, W, b2d)
